# baseline (speedup 1.0000x reference)
import functools

import jax
import jax.numpy as jnp
from jax import lax
from jax.experimental import pallas as pl
from jax.experimental.pallas import tpu as pltpu
from jax.experimental.pallas import tpu_sc as plsc

N = 10000
E = 320000
IN_CH = 128
HID = 8
HEADS = 8
F1 = HEADS * HID
F2 = 32
G1 = F1 + 16
G2 = F2 + 16

NC, NS, LANES = 2, 16, 16
NW = NC * NS
K = 128
CH = 80
EPW = CH * K
EPAD = NW * EPW
ETAIL = EPAD + 2 * K
NPAD = 10112
RPS = NPAD // NS

BN = 1000


def _leaky(v):
    return jnp.where(v >= 0, v, 0.2 * v)


def _make_edge_kernel(F, per_head):
    G = F + 16
    cpe_g = G // LANES

    mesh = plsc.VectorSubcoreMesh(core_axis_name="c", subcore_axis_name="s",
                                  num_cores=NC, num_subcores=NS)

    @functools.partial(
        pl.kernel,
        out_type=jax.ShapeDtypeStruct((NC, NPAD, G), jnp.float32),
        mesh=mesh,
        scratch_types=[
            pltpu.VMEM((CH + 2, K), jnp.int32),
            pltpu.VMEM((CH + 2, K), jnp.int32),
            pltpu.VMEM((K, G), jnp.float32),
            pltpu.VMEM((K, G), jnp.float32),
            pltpu.VMEM((K, 16), jnp.float32),
            pltpu.VMEM((K, 16), jnp.float32),
            pltpu.VMEM_SHARED((NPAD, G), jnp.float32),
            pltpu.SemaphoreType.DMA,
            pltpu.SemaphoreType.DMA,
            pltpu.SemaphoreType.DMA,
            pltpu.SemaphoreType.DMA,
        ],
        compiler_params=pltpu.CompilerParams(use_tc_tiling_on_sc=False),
    )
    def edge_kernel(src_hbm, dst_hbm, gtab, utab, acc_out,
                    src_v, dst_v, g0, g1, u0, u1, acc_s,
                    sg0, sg1, su0, su1):
        c = lax.axis_index("c")
        s = lax.axis_index("s")
        wid = s * NC + c
        zero16 = jnp.zeros((LANES,), jnp.float32)
        iot = lax.iota(jnp.int32, LANES)
        bufs = ((g0, u0, sg0, su0), (g1, u1, sg1, su1))

        pltpu.sync_copy(src_hbm.at[pl.ds(wid * CH, CH + 2)], src_v)
        pltpu.sync_copy(dst_hbm.at[pl.ds(wid * CH, CH + 2)], dst_v)

        @plsc.parallel_loop(0, K * cpe_g, unroll=4)
        def _zero(i):
            g0[i // cpe_g, pl.ds((i % cpe_g) * LANES, LANES)] = zero16

        for k in range((RPS + K - 1) // K):
            rows = min(K, RPS - k * K)
            pltpu.sync_copy(g0.at[pl.ds(0, rows)],
                            acc_s.at[pl.ds(s * RPS + k * K, rows)])
        plsc.subcore_barrier()

        def fire(b, chunk):
            g, u, sg, su = bufs[b]
            pltpu.async_copy(gtab.at[src_v.at[chunk]], g, sg)
            pltpu.async_copy(utab.at[dst_v.at[chunk]], u, su)

        def drain(b, chunk):
            g, u, sg, su = bufs[b]
            pltpu.make_async_copy(gtab.at[src_v.at[chunk]], g, sg).wait()
            pltpu.make_async_copy(utab.at[dst_v.at[chunk]], u, su).wait()

        for b in (0, 1):
            fire(b, b)

        @pl.loop(0, CH, step=2)
        def _chunks(j):
            for b in (0, 1):
                g, u, sg, su = bufs[b]
                drain(b, j + b)

                @plsc.parallel_loop(0, K, unroll=8)
                def _edge(e):
                    t = g[e, pl.ds(F, LANES)]
                    w16 = jnp.exp(_leaky(t + u[e]))
                    g[e, pl.ds(F, LANES)] = w16
                    for q in range(F // LANES):
                        if per_head:
                            idx = lax.shift_right_logical(iot, 3) + (2 * q)
                        else:
                            idx = iot & 0
                        wexp = lax.gather(
                            w16, idx[:, None],
                            lax.GatherDimensionNumbers(
                                offset_dims=(), collapsed_slice_dims=(0,),
                                start_index_map=(0,)),
                            slice_sizes=(1,),
                            mode=lax.GatherScatterMode.PROMISE_IN_BOUNDS)
                        sl = pl.ds(q * LANES, LANES)
                        g[e, sl] = g[e, sl] * wexp

                pltpu.sync_copy(g, acc_s.at[dst_v.at[j + b]], add=True)
                fire(b, j + b + 2)

        for b in (0, 1):
            drain(b, CH + b)

        plsc.subcore_barrier()
        pltpu.sync_copy(acc_s.at[pl.ds(s * RPS, RPS)],
                        acc_out.at[c, pl.ds(s * RPS, RPS)])

    return edge_kernel


@functools.lru_cache(maxsize=None)
def _edge_kernel(F, per_head):
    return _make_edge_kernel(F, per_head)


def _tc1_body(x_ref, w1_ref, mt_ref, mu_ref, g_ref, u_ref):
    h = jnp.dot(x_ref[...], w1_ref[...], preferred_element_type=jnp.float32)
    t = jnp.dot(h, mt_ref[...], preferred_element_type=jnp.float32)
    g_ref[...] = jnp.concatenate([h, t], axis=1)
    u_ref[...] = jnp.dot(h, mu_ref[...], preferred_element_type=jnp.float32)


def _tc1(x, W1, M1T, M1U):
    return pl.pallas_call(
        _tc1_body,
        grid=(N // BN,),
        in_specs=[
            pl.BlockSpec((BN, IN_CH), lambda i: (i, 0)),
            pl.BlockSpec((IN_CH, F1), lambda i: (0, 0)),
            pl.BlockSpec((F1, 16), lambda i: (0, 0)),
            pl.BlockSpec((F1, 16), lambda i: (0, 0)),
        ],
        out_specs=[
            pl.BlockSpec((BN, G1), lambda i: (i, 0)),
            pl.BlockSpec((BN, 16), lambda i: (i, 0)),
        ],
        out_shape=[
            jax.ShapeDtypeStruct((N, G1), jnp.float32),
            jax.ShapeDtypeStruct((N, 16), jnp.float32),
        ],
    )(x, W1, M1T, M1U)


def _tc2_body(acc_ref, g1_ref, r_ref, b1_ref, w2_ref, mt_ref, mu_ref,
              g2_ref, u2_ref):
    acc = acc_ref[...]
    g1 = g1_ref[...]
    h1 = g1[:, :F1]
    t1 = g1[:, F1:]
    wself = jnp.exp(_leaky(t1[:, :8] + t1[:, 8:]))
    r = r_ref[...]
    num = acc[0, :, :F1] + acc[1, :, :F1] + h1 * jnp.dot(
        wself, r, preferred_element_type=jnp.float32)
    den8 = acc[0, :, F1:F1 + 8] + acc[1, :, F1:F1 + 8] + wself
    den = jnp.dot(den8, r, preferred_element_type=jnp.float32)
    out1 = num / (den + 1e-16) + b1_ref[...]
    h2 = jnp.where(out1 > 0, out1, jnp.exp(out1) - 1.0)
    hh2 = jnp.dot(h2, w2_ref[...], preferred_element_type=jnp.float32)
    t2 = jnp.dot(hh2, mt_ref[...], preferred_element_type=jnp.float32)
    g2_ref[...] = jnp.concatenate([hh2, t2], axis=1)
    u2_ref[...] = jnp.dot(hh2, mu_ref[...], preferred_element_type=jnp.float32)


def _tc2(acc1, g1, R, b1, W2, M2T, M2U):
    return pl.pallas_call(
        _tc2_body,
        grid=(N // BN,),
        in_specs=[
            pl.BlockSpec((NC, BN, G1), lambda i: (0, i, 0)),
            pl.BlockSpec((BN, G1), lambda i: (i, 0)),
            pl.BlockSpec((HEADS, F1), lambda i: (0, 0)),
            pl.BlockSpec((1, F1), lambda i: (0, 0)),
            pl.BlockSpec((F1, F2), lambda i: (0, 0)),
            pl.BlockSpec((F2, 16), lambda i: (0, 0)),
            pl.BlockSpec((F2, 16), lambda i: (0, 0)),
        ],
        out_specs=[
            pl.BlockSpec((BN, G2), lambda i: (i, 0)),
            pl.BlockSpec((BN, 16), lambda i: (i, 0)),
        ],
        out_shape=[
            jax.ShapeDtypeStruct((N, G2), jnp.float32),
            jax.ShapeDtypeStruct((N, 16), jnp.float32),
        ],
    )(acc1, g1, R, b1, W2, M2T, M2U)


def _tc3_body(acc_ref, g2_ref, b2_ref, o_ref):
    acc = acc_ref[...]
    g2 = g2_ref[...]
    hh2 = g2[:, :F2]
    t2 = g2[:, F2:]
    w2 = jnp.exp(_leaky(t2[:, 0:1] + t2[:, 8:9]))
    num = acc[0, :, :F2] + acc[1, :, :F2] + hh2 * w2
    den = acc[0, :, F2:F2 + 1] + acc[1, :, F2:F2 + 1] + w2
    out = num / (den + 1e-16) + b2_ref[...]
    m = jnp.max(out, axis=1, keepdims=True)
    lse = jnp.log(jnp.sum(jnp.exp(out - m), axis=1, keepdims=True)) + m
    o_ref[...] = out - lse


def _tc3(acc2, g2, b2):
    return pl.pallas_call(
        _tc3_body,
        grid=(N // BN,),
        in_specs=[
            pl.BlockSpec((NC, BN, G2), lambda i: (0, i, 0)),
            pl.BlockSpec((BN, G2), lambda i: (i, 0)),
            pl.BlockSpec((1, F2), lambda i: (0, 0)),
        ],
        out_specs=pl.BlockSpec((BN, F2), lambda i: (i, 0)),
        out_shape=jax.ShapeDtypeStruct((N, F2), jnp.float32),
    )(acc2, g2, b2)


def kernel(x, edge_index, W1, att_src1, att_dst1, b1, W2, att_src2, att_dst2, b2):
    a1s = att_src1.reshape(HEADS, HID)
    a1d = att_dst1.reshape(HEADS, HID)
    eye = jnp.eye(HEADS, dtype=jnp.float32)
    Ms = (eye[:, None, :] * a1s[:, :, None]).reshape(F1, HEADS)
    Md = (eye[:, None, :] * a1d[:, :, None]).reshape(F1, HEADS)
    M1T = jnp.concatenate([Ms, Md], axis=1)
    M1U = jnp.concatenate([Md, Ms], axis=1)
    a2s = att_src2.reshape(F2, 1)
    a2d = att_dst2.reshape(F2, 1)
    M2T = jnp.concatenate([jnp.tile(a2s, (1, 8)), jnp.tile(a2d, (1, 8))], axis=1)
    M2U = jnp.concatenate([jnp.tile(a2d, (1, 8)), jnp.tile(a2s, (1, 8))], axis=1)
    R = jnp.repeat(jnp.eye(HEADS, dtype=jnp.float32), HID, axis=1)

    src = jnp.concatenate(
        [edge_index[0], jnp.zeros((ETAIL - E,), jnp.int32)]
    ).reshape(NW * CH + 2, K)
    dst = jnp.concatenate(
        [edge_index[1], jnp.full((ETAIL - E,), N, jnp.int32)]
    ).reshape(NW * CH + 2, K)

    g1, U1 = _tc1(x, W1, M1T, M1U)
    U1p = jnp.concatenate([U1, jnp.zeros((NPAD - N, 16), jnp.float32)], axis=0)
    acc1 = _edge_kernel(F1, True)(src, dst, g1, U1p)
    g2, U2 = _tc2(acc1, g1, R, b1.reshape(1, F1), W2, M2T, M2U)
    U2p = jnp.concatenate([U2, jnp.zeros((NPAD - N, 16), jnp.float32)], axis=0)
    acc2 = _edge_kernel(F2, False)(src, dst, g2, U2p)
    return _tc3(acc2, g2, b2.reshape(1, F2))


# speedup vs baseline: 1.0065x; 1.0065x over previous
import functools

import jax
import jax.numpy as jnp
from jax import lax
from jax.experimental import pallas as pl
from jax.experimental.pallas import tpu as pltpu
from jax.experimental.pallas import tpu_sc as plsc

N = 10000
E = 320000
IN_CH = 128
HID = 8
HEADS = 8
F1 = HEADS * HID
F2 = 32
G1 = F1 + 16
G2 = F2 + 16

NC, NS, LANES = 2, 16, 16
NW = NC * NS
K = 128
CH = 80
EPW = CH * K
EPAD = NW * EPW
ETAIL = EPAD + 2 * K
NPAD = 10112
RPS = NPAD // NS

BN = 1000


def _leaky(v):
    return jnp.where(v >= 0, v, 0.2 * v)


def _make_edge_kernel(F, per_head):
    G = F + 16
    cpe_g = G // LANES

    mesh = plsc.VectorSubcoreMesh(core_axis_name="c", subcore_axis_name="s",
                                  num_cores=NC, num_subcores=NS)

    @functools.partial(
        pl.kernel,
        out_type=jax.ShapeDtypeStruct((NC, NPAD, G), jnp.float32),
        mesh=mesh,
        scratch_types=[
            pltpu.VMEM((CH + 2, K), jnp.int32),
            pltpu.VMEM((CH + 2, K), jnp.int32),
            pltpu.VMEM((K, G), jnp.float32),
            pltpu.VMEM((K, G), jnp.float32),
            pltpu.VMEM((K, G), jnp.float32),
            pltpu.VMEM((K, G), jnp.float32),
            pltpu.VMEM((K, 16), jnp.float32),
            pltpu.VMEM((K, 16), jnp.float32),
            pltpu.VMEM((K, 16), jnp.float32),
            pltpu.VMEM((K, 16), jnp.float32),
            pltpu.VMEM_SHARED((NPAD, G), jnp.float32),
        ] + [pltpu.SemaphoreType.DMA] * 12,
        compiler_params=pltpu.CompilerParams(use_tc_tiling_on_sc=False),
    )
    def edge_kernel(src_hbm, dst_hbm, gtab, utab, acc_out,
                    src_v, dst_v, g0, g1, g2, g3, u0, u1, u2, u3, acc_s,
                    sg0, sg1, sg2, sg3, su0, su1, su2, su3,
                    ss0, ss1, ss2, ss3):
        c = lax.axis_index("c")
        s = lax.axis_index("s")
        wid = s * NC + c
        zero16 = jnp.zeros((LANES,), jnp.float32)
        iot = lax.iota(jnp.int32, LANES)
        bufs = ((g0, u0, sg0, su0, ss0), (g1, u1, sg1, su1, ss1),
                (g2, u2, sg2, su2, ss2), (g3, u3, sg3, su3, ss3))

        pltpu.sync_copy(src_hbm.at[pl.ds(wid * CH, CH + 2)], src_v)
        pltpu.sync_copy(dst_hbm.at[pl.ds(wid * CH, CH + 2)], dst_v)

        @plsc.parallel_loop(0, K * cpe_g, unroll=4)
        def _zero(i):
            g0[i // cpe_g, pl.ds((i % cpe_g) * LANES, LANES)] = zero16

        for k in range((RPS + K - 1) // K):
            rows = min(K, RPS - k * K)
            pltpu.sync_copy(g0.at[pl.ds(0, rows)],
                            acc_s.at[pl.ds(s * RPS + k * K, rows)])
        plsc.subcore_barrier()

        def fire(b, chunk):
            g, u, sg, su, ss = bufs[b]
            pltpu.async_copy(gtab.at[src_v.at[chunk]], g, sg)
            pltpu.async_copy(utab.at[dst_v.at[chunk]], u, su)

        def drain(b, chunk):
            g, u, sg, su, ss = bufs[b]
            pltpu.make_async_copy(gtab.at[src_v.at[chunk]], g, sg).wait()
            pltpu.make_async_copy(utab.at[dst_v.at[chunk]], u, su).wait()

        def drain_scatter(b, chunk):
            g, u, sg, su, ss = bufs[b]
            pltpu.make_async_copy(g, acc_s.at[dst_v.at[chunk]], ss).wait()

        for b in (0, 1):
            fire(b, b)

        @pl.loop(0, CH, step=4)
        def _chunks(j):
            for b in (0, 1, 2, 3):
                g, u, sg, su, ss = bufs[b]
                drain(b, j + b)

                @plsc.parallel_loop(0, K, unroll=8)
                def _edge(e):
                    t = g[e, pl.ds(F, LANES)]
                    w16 = jnp.exp(_leaky(t + u[e]))
                    g[e, pl.ds(F, LANES)] = w16
                    for q in range(F // LANES):
                        if per_head:
                            idx = lax.shift_right_logical(iot, 3) + (2 * q)
                        else:
                            idx = iot & 0
                        wexp = lax.gather(
                            w16, idx[:, None],
                            lax.GatherDimensionNumbers(
                                offset_dims=(), collapsed_slice_dims=(0,),
                                start_index_map=(0,)),
                            slice_sizes=(1,),
                            mode=lax.GatherScatterMode.PROMISE_IN_BOUNDS)
                        sl = pl.ds(q * LANES, LANES)
                        g[e, sl] = g[e, sl] * wexp

                pltpu.async_copy(g, acc_s.at[dst_v.at[j + b]], ss, add=True)
                bn = (b + 2) % 4
                if b < 2:
                    @pl.when(j > 0)
                    def _wait_prev():
                        drain_scatter(bn, j + b - 2)
                else:
                    drain_scatter(bn, j + b - 2)
                fire(bn, j + b + 2)

        drain(0, CH)
        drain(1, CH + 1)
        drain_scatter(2, CH - 2)
        drain_scatter(3, CH - 1)

        plsc.subcore_barrier()
        pltpu.sync_copy(acc_s.at[pl.ds(s * RPS, RPS)],
                        acc_out.at[c, pl.ds(s * RPS, RPS)])

    return edge_kernel


@functools.lru_cache(maxsize=None)
def _edge_kernel(F, per_head):
    return _make_edge_kernel(F, per_head)


def _tc1_body(x_ref, w1_ref, mt_ref, mu_ref, g_ref, u_ref):
    h = jnp.dot(x_ref[...], w1_ref[...], preferred_element_type=jnp.float32)
    t = jnp.dot(h, mt_ref[...], preferred_element_type=jnp.float32)
    g_ref[...] = jnp.concatenate([h, t], axis=1)
    u_ref[...] = jnp.dot(h, mu_ref[...], preferred_element_type=jnp.float32)


def _tc1(x, W1, M1T, M1U):
    return pl.pallas_call(
        _tc1_body,
        grid=(N // BN,),
        in_specs=[
            pl.BlockSpec((BN, IN_CH), lambda i: (i, 0)),
            pl.BlockSpec((IN_CH, F1), lambda i: (0, 0)),
            pl.BlockSpec((F1, 16), lambda i: (0, 0)),
            pl.BlockSpec((F1, 16), lambda i: (0, 0)),
        ],
        out_specs=[
            pl.BlockSpec((BN, G1), lambda i: (i, 0)),
            pl.BlockSpec((BN, 16), lambda i: (i, 0)),
        ],
        out_shape=[
            jax.ShapeDtypeStruct((N, G1), jnp.float32),
            jax.ShapeDtypeStruct((N, 16), jnp.float32),
        ],
    )(x, W1, M1T, M1U)


def _tc2_body(acc_ref, g1_ref, r_ref, b1_ref, w2_ref, mt_ref, mu_ref,
              g2_ref, u2_ref):
    acc = acc_ref[...]
    g1 = g1_ref[...]
    h1 = g1[:, :F1]
    t1 = g1[:, F1:]
    wself = jnp.exp(_leaky(t1[:, :8] + t1[:, 8:]))
    r = r_ref[...]
    num = acc[0, :, :F1] + acc[1, :, :F1] + h1 * jnp.dot(
        wself, r, preferred_element_type=jnp.float32)
    den8 = acc[0, :, F1:F1 + 8] + acc[1, :, F1:F1 + 8] + wself
    den = jnp.dot(den8, r, preferred_element_type=jnp.float32)
    out1 = num / (den + 1e-16) + b1_ref[...]
    h2 = jnp.where(out1 > 0, out1, jnp.exp(out1) - 1.0)
    hh2 = jnp.dot(h2, w2_ref[...], preferred_element_type=jnp.float32)
    t2 = jnp.dot(hh2, mt_ref[...], preferred_element_type=jnp.float32)
    g2_ref[...] = jnp.concatenate([hh2, t2], axis=1)
    u2_ref[...] = jnp.dot(hh2, mu_ref[...], preferred_element_type=jnp.float32)


def _tc2(acc1, g1, R, b1, W2, M2T, M2U):
    return pl.pallas_call(
        _tc2_body,
        grid=(N // BN,),
        in_specs=[
            pl.BlockSpec((NC, BN, G1), lambda i: (0, i, 0)),
            pl.BlockSpec((BN, G1), lambda i: (i, 0)),
            pl.BlockSpec((HEADS, F1), lambda i: (0, 0)),
            pl.BlockSpec((1, F1), lambda i: (0, 0)),
            pl.BlockSpec((F1, F2), lambda i: (0, 0)),
            pl.BlockSpec((F2, 16), lambda i: (0, 0)),
            pl.BlockSpec((F2, 16), lambda i: (0, 0)),
        ],
        out_specs=[
            pl.BlockSpec((BN, G2), lambda i: (i, 0)),
            pl.BlockSpec((BN, 16), lambda i: (i, 0)),
        ],
        out_shape=[
            jax.ShapeDtypeStruct((N, G2), jnp.float32),
            jax.ShapeDtypeStruct((N, 16), jnp.float32),
        ],
    )(acc1, g1, R, b1, W2, M2T, M2U)


def _tc3_body(acc_ref, g2_ref, b2_ref, o_ref):
    acc = acc_ref[...]
    g2 = g2_ref[...]
    hh2 = g2[:, :F2]
    t2 = g2[:, F2:]
    w2 = jnp.exp(_leaky(t2[:, 0:1] + t2[:, 8:9]))
    num = acc[0, :, :F2] + acc[1, :, :F2] + hh2 * w2
    den = acc[0, :, F2:F2 + 1] + acc[1, :, F2:F2 + 1] + w2
    out = num / (den + 1e-16) + b2_ref[...]
    m = jnp.max(out, axis=1, keepdims=True)
    lse = jnp.log(jnp.sum(jnp.exp(out - m), axis=1, keepdims=True)) + m
    o_ref[...] = out - lse


def _tc3(acc2, g2, b2):
    return pl.pallas_call(
        _tc3_body,
        grid=(N // BN,),
        in_specs=[
            pl.BlockSpec((NC, BN, G2), lambda i: (0, i, 0)),
            pl.BlockSpec((BN, G2), lambda i: (i, 0)),
            pl.BlockSpec((1, F2), lambda i: (0, 0)),
        ],
        out_specs=pl.BlockSpec((BN, F2), lambda i: (i, 0)),
        out_shape=jax.ShapeDtypeStruct((N, F2), jnp.float32),
    )(acc2, g2, b2)


def kernel(x, edge_index, W1, att_src1, att_dst1, b1, W2, att_src2, att_dst2, b2):
    a1s = att_src1.reshape(HEADS, HID)
    a1d = att_dst1.reshape(HEADS, HID)
    eye = jnp.eye(HEADS, dtype=jnp.float32)
    Ms = (eye[:, None, :] * a1s[:, :, None]).reshape(F1, HEADS)
    Md = (eye[:, None, :] * a1d[:, :, None]).reshape(F1, HEADS)
    M1T = jnp.concatenate([Ms, Md], axis=1)
    M1U = jnp.concatenate([Md, Ms], axis=1)
    a2s = att_src2.reshape(F2, 1)
    a2d = att_dst2.reshape(F2, 1)
    M2T = jnp.concatenate([jnp.tile(a2s, (1, 8)), jnp.tile(a2d, (1, 8))], axis=1)
    M2U = jnp.concatenate([jnp.tile(a2d, (1, 8)), jnp.tile(a2s, (1, 8))], axis=1)
    R = jnp.repeat(jnp.eye(HEADS, dtype=jnp.float32), HID, axis=1)

    src = jnp.concatenate(
        [edge_index[0], jnp.zeros((ETAIL - E,), jnp.int32)]
    ).reshape(NW * CH + 2, K)
    dst = jnp.concatenate(
        [edge_index[1], jnp.full((ETAIL - E,), N, jnp.int32)]
    ).reshape(NW * CH + 2, K)

    g1, U1 = _tc1(x, W1, M1T, M1U)
    U1p = jnp.concatenate([U1, jnp.zeros((NPAD - N, 16), jnp.float32)], axis=0)
    acc1 = _edge_kernel(F1, True)(src, dst, g1, U1p)
    g2, U2 = _tc2(acc1, g1, R, b1.reshape(1, F1), W2, M2T, M2U)
    U2p = jnp.concatenate([U2, jnp.zeros((NPAD - N, 16), jnp.float32)], axis=0)
    acc2 = _edge_kernel(F2, False)(src, dst, g2, U2p)
    return _tc3(acc2, g2, b2.reshape(1, F2))
